# baseline (speedup 1.0000x reference)
import functools

import jax
import jax.numpy as jnp
from jax import lax
from jax.experimental import pallas as pl
from jax.experimental.pallas import tpu as pltpu
from jax.experimental.pallas import tpu_sc as plsc

NC = 2
NS = 16
NW = NC * NS
CH = 128


def _cdiv(a, b):
    return (a + b - 1) // b


CH1 = 500
C1 = 80
SEG = 20


def _make_scatter(n_pad, dh):
    mesh = plsc.VectorSubcoreMesh(core_axis_name="c", subcore_axis_name="s")

    @functools.partial(
        pl.kernel,
        out_type=jax.ShapeDtypeStruct((NC * n_pad, dh), jnp.float32),
        mesh=mesh,
        scratch_types=[
            pltpu.VMEM((SEG, CH1), jnp.int32),
            pltpu.VMEM((SEG, CH1), jnp.int32),
            [pltpu.VMEM((CH1, dh), jnp.float32) for _ in range(2)],
            pltpu.VMEM_SHARED((n_pad, dh), jnp.float32),
            [pltpu.SemaphoreType.DMA for _ in range(2)],
            [pltpu.SemaphoreType.DMA for _ in range(2)],
        ],
        compiler_params=pltpu.CompilerParams(use_tc_tiling_on_sc=False),
    )
    def scatter_k(x_hbm, gidx_hbm, sidx_hbm, zeros_hbm, out_hbm,
                  gidx_v, sidx_v, rows_v, acc_sh, gsem, asem):
        cid = lax.axis_index("c")
        sid = lax.axis_index("s")
        rpt = n_pad // NS
        row0 = pl.multiple_of(sid * rpt, 8)
        pltpu.sync_copy(zeros_hbm.at[pl.ds(row0, rpt)],
                        acc_sh.at[pl.ds(row0, rpt)])
        plsc.subcore_barrier()

        def body(g, carry):
            ad = [None, None]
            for bi in range(2):
                ch = g * 2 + bi
                pltpu.async_copy(x_hbm.at[gidx_v.at[ch]], rows_v[bi],
                                 gsem[bi]).wait()
                ad[bi] = pltpu.async_copy(rows_v[bi],
                                          acc_sh.at[sidx_v.at[ch]],
                                          asem[bi], add=True)
            ad[0].wait()
            ad[1].wait()
            return carry

        for seg in range(C1 // SEG):
            grow = cid * NS * C1 + sid * C1 + seg * SEG
            srow = sid * C1 + seg * SEG
            pltpu.sync_copy(gidx_hbm.at[pl.ds(grow, SEG)], gidx_v)
            pltpu.sync_copy(sidx_hbm.at[pl.ds(srow, SEG)], sidx_v)
            lax.fori_loop(0, SEG // 2, body, 0)
        plsc.subcore_barrier()
        pltpu.sync_copy(acc_sh.at[pl.ds(row0, rpt)],
                        out_hbm.at[pl.ds(pl.multiple_of(cid * n_pad + sid * rpt, 8),
                                         rpt)])

    return scatter_k


CH2 = 400
C2A = 25


def _make_gather(n_pad, d, n_edges):
    mesh = plsc.VectorSubcoreMesh(core_axis_name="c", subcore_axis_name="s")

    @functools.partial(
        pl.kernel,
        out_type=jax.ShapeDtypeStruct((n_edges, d), jnp.float32),
        mesh=mesh,
        scratch_types=[
            pltpu.VMEM((C2A, CH2), jnp.int32),
            pltpu.VMEM((C2A, CH2), jnp.int32),
            [pltpu.VMEM((CH2, d), jnp.float32) for _ in range(2)],
            [pltpu.SemaphoreType.DMA for _ in range(2)],
            [pltpu.SemaphoreType.DMA for _ in range(2)],
            [pltpu.SemaphoreType.DMA for _ in range(2)],
        ],
        compiler_params=pltpu.CompilerParams(use_tc_tiling_on_sc=False),
    )
    def gather_k(y1_hbm, y2_hbm, e_hbm, sidx_hbm, ridx_hbm, out_hbm,
                 sidx_v, ridx_v, buf_v, sem, sem2, stsem):
        cid = lax.axis_index("c")
        sid = lax.axis_index("s")
        wid = cid * NS + sid
        pltpu.sync_copy(sidx_hbm.at[wid], sidx_v)
        pltpu.sync_copy(ridx_hbm.at[wid], ridx_v)
        base = wid * C2A * CH2

        def body(g, carry):
            st = [None, None]
            for bi in range(2):
                ch = g * 2 + bi
                rows = pl.ds(base + ch * CH2, CH2)
                pltpu.sync_copy(e_hbm.at[rows], buf_v[bi])
                pltpu.async_copy(y1_hbm.at[sidx_v.at[ch]], buf_v[bi],
                                 sem[bi], add=True).wait()
                pltpu.async_copy(y2_hbm.at[ridx_v.at[ch]], buf_v[bi],
                                 sem2[bi], add=True).wait()
                st[bi] = pltpu.async_copy(buf_v[bi], out_hbm.at[rows],
                                          stsem[bi])
            st[0].wait()
            st[1].wait()
            return carry

        lax.fori_loop(0, C2A // 2, body, 0)

        ch = C2A - 1
        rows = pl.ds(base + ch * CH2, CH2)
        pltpu.sync_copy(e_hbm.at[rows], buf_v[0])
        pltpu.async_copy(y1_hbm.at[sidx_v.at[ch]], buf_v[0],
                         sem[0], add=True).wait()
        pltpu.async_copy(y2_hbm.at[ridx_v.at[ch]], buf_v[0],
                         sem2[0], add=True).wait()
        pltpu.sync_copy(buf_v[0], out_hbm.at[rows])

    return gather_k


def _node_matmul(agg, w1a, w1b, w2a, w2b, n_pad, dh, d):
    br = n_pad // 8
    grid = (8,)

    def body(pa_ref, pb_ref, w1a_ref, w1b_ref, w2a_ref, w2b_ref,
             y1_ref, y2_ref):
        pa = pa_ref[...]
        pb = pb_ref[...]
        y1_ref[...] = (jnp.dot(pa, w1a_ref[...], preferred_element_type=jnp.float32)
                       + jnp.dot(pb, w1b_ref[...], preferred_element_type=jnp.float32))
        y2_ref[...] = (jnp.dot(pa, w2a_ref[...], preferred_element_type=jnp.float32)
                       + jnp.dot(pb, w2b_ref[...], preferred_element_type=jnp.float32))

    wspec = pl.BlockSpec((dh, d), lambda i: (0, 0))
    y1, y2 = pl.pallas_call(
        body,
        grid=grid,
        in_specs=[
            pl.BlockSpec((br, dh), lambda i: (i, 0)),
            pl.BlockSpec((br, dh), lambda i: (i + 8, 0)),
            wspec, wspec, wspec, wspec,
        ],
        out_specs=[
            pl.BlockSpec((br, d), lambda i: (i, 0)),
            pl.BlockSpec((br, d), lambda i: (i, 0)),
        ],
        out_shape=[
            jax.ShapeDtypeStruct((n_pad, d), jnp.float32),
            jax.ShapeDtypeStruct((n_pad, d), jnp.float32),
        ],
    )(agg, agg, w1a, w1b, w2a, w2b)
    return y1, y2


def _edge_mlp(edge_attr, w3, b, n_edges, d_edge, d):
    br = 1280
    grid = (n_edges // br,)

    def body(ea_ref, w3_ref, b_ref, o_ref):
        o_ref[...] = (jnp.dot(ea_ref[...], w3_ref[...],
                              preferred_element_type=jnp.float32)
                      + b_ref[...])

    return pl.pallas_call(
        body,
        grid=grid,
        in_specs=[
            pl.BlockSpec((br, d_edge), lambda i: (i, 0)),
            pl.BlockSpec((d_edge, d), lambda i: (0, 0)),
            pl.BlockSpec((1, d), lambda i: (0, 0)),
        ],
        out_specs=pl.BlockSpec((br, d), lambda i: (i, 0)),
        out_shape=jax.ShapeDtypeStruct((n_edges, d), jnp.float32),
    )(edge_attr, w3, b)


def kernel(x, edge_index, edge_attr, W, b):
    n_nodes, d = x.shape
    n_edges, d_edge = edge_attr.shape

    dh = d // 2
    s = edge_index[0].astype(jnp.int32)
    r = edge_index[1].astype(jnp.int32)

    gidx = jnp.concatenate([r, s])
    gall = jnp.concatenate([gidx, gidx + n_nodes]).reshape(NC * NS * C1, CH1)
    sall = jnp.concatenate([s, r]).reshape(NS * C1, CH1)
    xcat = jnp.concatenate([x[:, :dh], x[:, dh:]], axis=0)

    n_pad = 128 * _cdiv(n_nodes, 128)
    zeros = jnp.zeros((n_pad, dh), jnp.float32)

    agg = _make_scatter(n_pad, dh)(xcat, gall, sall, zeros)

    y1, y2 = _node_matmul(agg, W[:dh], W[dh:d], W[d:d + dh], W[d + dh:2 * d],
                          n_pad, dh, d)

    e_term = _edge_mlp(edge_attr, W[2 * d:], b.reshape(1, d),
                       n_edges, d_edge, d)

    return _make_gather(n_pad, d, n_edges)(y1, y2, e_term,
                                           s.reshape(NW, C2A, CH2),
                                           r.reshape(NW, C2A, CH2))


# speedup vs baseline: 1.0063x; 1.0063x over previous
import functools

import jax
import jax.numpy as jnp
from jax import lax
from jax.experimental import pallas as pl
from jax.experimental.pallas import tpu as pltpu
from jax.experimental.pallas import tpu_sc as plsc

NC = 2
NS = 16
NW = NC * NS


def _cdiv(a, b):
    return (a + b - 1) // b


CH1 = 500
C1 = 80
SEG = 40


def _make_scatter(n_pad, dh):
    mesh = plsc.VectorSubcoreMesh(core_axis_name="c", subcore_axis_name="s")

    @functools.partial(
        pl.kernel,
        out_type=jax.ShapeDtypeStruct((NC * n_pad, dh), jnp.float32),
        mesh=mesh,
        scratch_types=[
            pltpu.VMEM((SEG, CH1), jnp.int32),
            pltpu.VMEM((SEG, CH1), jnp.int32),
            pltpu.VMEM((CH1, dh), jnp.float32),
            pltpu.VMEM_SHARED((n_pad, dh), jnp.float32),
            pltpu.SemaphoreType.DMA,
        ],
        compiler_params=pltpu.CompilerParams(use_tc_tiling_on_sc=False),
    )
    def scatter_k(x_hbm, gidx_hbm, sidx_hbm, zeros_hbm, out_hbm,
                  gidx_v, sidx_v, rows_v, acc_sh, gsem):
        cid = lax.axis_index("c")
        sid = lax.axis_index("s")
        rpt = n_pad // NS
        row0 = pl.multiple_of(sid * rpt, 8)
        pltpu.sync_copy(zeros_hbm.at[pl.ds(row0, rpt)],
                        acc_sh.at[pl.ds(row0, rpt)])
        plsc.subcore_barrier()

        def body(ch, carry):
            pltpu.async_copy(x_hbm.at[gidx_v.at[ch]], rows_v, gsem).wait()
            pltpu.sync_copy(rows_v, acc_sh.at[sidx_v.at[ch]], add=True)
            return carry

        for seg in range(C1 // SEG):
            grow = cid * NS * C1 + sid * C1 + seg * SEG
            srow = sid * C1 + seg * SEG
            pltpu.sync_copy(gidx_hbm.at[pl.ds(grow, SEG)], gidx_v)
            pltpu.sync_copy(sidx_hbm.at[pl.ds(srow, SEG)], sidx_v)
            lax.fori_loop(0, SEG, body, 0)
        plsc.subcore_barrier()
        pltpu.sync_copy(acc_sh.at[pl.ds(row0, rpt)],
                        out_hbm.at[pl.ds(pl.multiple_of(cid * n_pad + sid * rpt, 8),
                                         rpt)])

    return scatter_k


CH2 = 400
C2A = 25


def _make_gather(n_pad, d, n_edges):
    mesh = plsc.VectorSubcoreMesh(core_axis_name="c", subcore_axis_name="s")

    @functools.partial(
        pl.kernel,
        out_type=jax.ShapeDtypeStruct((n_edges, d), jnp.float32),
        mesh=mesh,
        scratch_types=[
            pltpu.VMEM((C2A, CH2), jnp.int32),
            pltpu.VMEM((C2A, CH2), jnp.int32),
            [pltpu.VMEM((CH2, d), jnp.float32) for _ in range(2)],
            [pltpu.SemaphoreType.DMA for _ in range(2)],
            [pltpu.SemaphoreType.DMA for _ in range(2)],
            [pltpu.SemaphoreType.DMA for _ in range(2)],
        ],
        compiler_params=pltpu.CompilerParams(use_tc_tiling_on_sc=False),
    )
    def gather_k(y1_hbm, y2_hbm, e_hbm, sidx_hbm, ridx_hbm, out_hbm,
                 sidx_v, ridx_v, buf_v, sem, sem2, stsem):
        cid = lax.axis_index("c")
        sid = lax.axis_index("s")
        wid = cid * NS + sid
        pltpu.sync_copy(sidx_hbm.at[wid], sidx_v)
        pltpu.sync_copy(ridx_hbm.at[wid], ridx_v)
        base = wid * C2A * CH2

        def body(g, carry):
            st = [None, None]
            for bi in range(2):
                ch = g * 2 + bi
                rows = pl.ds(base + ch * CH2, CH2)
                pltpu.sync_copy(e_hbm.at[rows], buf_v[bi])
                pltpu.async_copy(y1_hbm.at[sidx_v.at[ch]], buf_v[bi],
                                 sem[bi], add=True).wait()
                pltpu.async_copy(y2_hbm.at[ridx_v.at[ch]], buf_v[bi],
                                 sem2[bi], add=True).wait()
                st[bi] = pltpu.async_copy(buf_v[bi], out_hbm.at[rows],
                                          stsem[bi])
            st[0].wait()
            st[1].wait()
            return carry

        lax.fori_loop(0, C2A // 2, body, 0)

        ch = C2A - 1
        rows = pl.ds(base + ch * CH2, CH2)
        pltpu.sync_copy(e_hbm.at[rows], buf_v[0])
        pltpu.async_copy(y1_hbm.at[sidx_v.at[ch]], buf_v[0],
                         sem[0], add=True).wait()
        pltpu.async_copy(y2_hbm.at[ridx_v.at[ch]], buf_v[0],
                         sem2[0], add=True).wait()
        pltpu.sync_copy(buf_v[0], out_hbm.at[rows])

    return gather_k


def _node_matmul(agg, w1a, w1b, w2a, w2b, n_pad, dh, d):
    br = n_pad // 8
    grid = (8,)

    def body(pa_ref, pb_ref, w1a_ref, w1b_ref, w2a_ref, w2b_ref,
             y1_ref, y2_ref):
        pa = pa_ref[...]
        pb = pb_ref[...]
        y1_ref[...] = (jnp.dot(pa, w1a_ref[...], preferred_element_type=jnp.float32)
                       + jnp.dot(pb, w1b_ref[...], preferred_element_type=jnp.float32))
        y2_ref[...] = (jnp.dot(pa, w2a_ref[...], preferred_element_type=jnp.float32)
                       + jnp.dot(pb, w2b_ref[...], preferred_element_type=jnp.float32))

    wspec = pl.BlockSpec((dh, d), lambda i: (0, 0))
    y1, y2 = pl.pallas_call(
        body,
        grid=grid,
        in_specs=[
            pl.BlockSpec((br, dh), lambda i: (i, 0)),
            pl.BlockSpec((br, dh), lambda i: (i + 8, 0)),
            wspec, wspec, wspec, wspec,
        ],
        out_specs=[
            pl.BlockSpec((br, d), lambda i: (i, 0)),
            pl.BlockSpec((br, d), lambda i: (i, 0)),
        ],
        out_shape=[
            jax.ShapeDtypeStruct((n_pad, d), jnp.float32),
            jax.ShapeDtypeStruct((n_pad, d), jnp.float32),
        ],
    )(agg, agg, w1a, w1b, w2a, w2b)
    return y1, y2


def _edge_mlp(edge_attr, w3, b, n_edges, d_edge, d):
    br = 1280
    grid = (n_edges // br,)

    def body(ea_ref, w3_ref, b_ref, o_ref):
        o_ref[...] = (jnp.dot(ea_ref[...], w3_ref[...],
                              preferred_element_type=jnp.float32)
                      + b_ref[...])

    return pl.pallas_call(
        body,
        grid=grid,
        in_specs=[
            pl.BlockSpec((br, d_edge), lambda i: (i, 0)),
            pl.BlockSpec((d_edge, d), lambda i: (0, 0)),
            pl.BlockSpec((1, d), lambda i: (0, 0)),
        ],
        out_specs=pl.BlockSpec((br, d), lambda i: (i, 0)),
        out_shape=jax.ShapeDtypeStruct((n_edges, d), jnp.float32),
    )(edge_attr, w3, b)


def kernel(x, edge_index, edge_attr, W, b):
    n_nodes, d = x.shape
    n_edges, d_edge = edge_attr.shape

    dh = d // 2
    s = edge_index[0].astype(jnp.int32)
    r = edge_index[1].astype(jnp.int32)

    gidx = jnp.concatenate([r, s])
    gall = jnp.concatenate([gidx, gidx + n_nodes]).reshape(NC * NS * C1, CH1)
    sall = jnp.concatenate([s, r]).reshape(NS * C1, CH1)
    xcat = jnp.concatenate([x[:, :dh], x[:, dh:]], axis=0)

    n_pad = 128 * _cdiv(n_nodes, 128)
    zeros = jnp.zeros((n_pad, dh), jnp.float32)

    agg = _make_scatter(n_pad, dh)(xcat, gall, sall, zeros)

    y1, y2 = _node_matmul(agg, W[:dh], W[dh:d], W[d:d + dh], W[d + dh:2 * d],
                          n_pad, dh, d)

    e_term = _edge_mlp(edge_attr, W[2 * d:], b.reshape(1, d),
                       n_edges, d_edge, d)

    return _make_gather(n_pad, d, n_edges)(y1, y2, e_term,
                                           s.reshape(NW, C2A, CH2),
                                           r.reshape(NW, C2A, CH2))
